# baseline (speedup 1.0000x reference)
import functools

import jax
import jax.numpy as jnp
from jax import lax
from jax.experimental import pallas as pl
from jax.experimental.pallas import tpu as pltpu
from jax.experimental.pallas import tpu_sc as plsc

_B, _N, _D = 64, 576, 768
_TT = _B * _N
_NC, _NS = 2, 16
_NW = _NC * _NS
_TPW = _TT // _NW
_CH = 32
_NCHUNK = _TPW // _CH
_LANES = 16
_MAGIC = 8388608.0


def _rne(v):
    return (v + _MAGIC) - _MAGIC


def _sc_body(x_hbm, rpf_hbm, rpt_hbm, cpf_hbm, cpt_hbm, rtab_hbm, ctab_hbm,
             out_hbm, pf_v, pt_v, qf_v, qt_v, ridx_v, cidx_v, in_v, rbuf,
             cbuf, sem):
    c = lax.axis_index("c")
    s = lax.axis_index("s")
    base = (c * _NS + s) * _TPW

    def chunk_body(k_i, carry):
        tok0 = base + k_i * _CH
        pltpu.sync_copy(rpf_hbm.at[pl.ds(tok0, _CH)], pf_v)
        pltpu.sync_copy(rpt_hbm.at[pl.ds(tok0, _CH)], pt_v)
        pltpu.sync_copy(cpf_hbm.at[pl.ds(tok0, _CH)], qf_v)
        pltpu.sync_copy(cpt_hbm.at[pl.ds(tok0, _CH)], qt_v)
        for g in range(_CH // _LANES):
            sl = pl.ds(g * _LANES, _LANES)
            rf = _rne(pf_v[sl] * 128.0)
            rt = _rne(pt_v[sl] * 128.0)
            cf = _rne(qf_v[sl] * 128.0)
            ct = _rne(qt_v[sl] * 128.0)
            ridx_v[sl] = _rne((rf + rt) * 0.5).astype(jnp.int32)
            cidx_v[sl] = _rne((cf + ct) * 0.5).astype(jnp.int32)
        pltpu.sync_copy(x_hbm.at[pl.ds(tok0 * _D, _CH * _D)], in_v)
        pltpu.async_copy(rtab_hbm.at[ridx_v], rbuf, sem).wait()
        pltpu.async_copy(ctab_hbm.at[cidx_v], cbuf, sem).wait()

        def tok_body(t, carry2):
            def vec_body(j, carry3):
                off = j * _LANES
                sl = pl.ds(off, _LANES)
                acc = (in_v[pl.ds(t * _D + off, _LANES)]
                       + rbuf[t, sl] + cbuf[t, sl])
                in_v[pl.ds(t * _D + off, _LANES)] = acc
                return carry3

            return lax.fori_loop(0, _D // _LANES, vec_body, carry2,
                                 unroll=8)

        lax.fori_loop(0, _CH, tok_body, 0)
        pltpu.sync_copy(in_v, out_hbm.at[pl.ds(tok0 * _D, _CH * _D)])
        return carry

    lax.fori_loop(0, _NCHUNK, chunk_body, 0)


_sc_call = functools.partial(
    pl.kernel,
    out_type=jax.ShapeDtypeStruct((_TT * _D,), jnp.float32),
    mesh=plsc.VectorSubcoreMesh(
        core_axis_name="c", subcore_axis_name="s",
        num_cores=_NC, num_subcores=_NS),
    scratch_types=[
        pltpu.VMEM((_CH,), jnp.float32),
        pltpu.VMEM((_CH,), jnp.float32),
        pltpu.VMEM((_CH,), jnp.float32),
        pltpu.VMEM((_CH,), jnp.float32),
        pltpu.VMEM((_CH,), jnp.int32),
        pltpu.VMEM((_CH,), jnp.int32),
        pltpu.VMEM((_CH * _D,), jnp.float32),
        pltpu.VMEM((_CH, _D), jnp.float32),
        pltpu.VMEM((_CH, _D), jnp.float32),
        pltpu.SemaphoreType.DMA,
    ],
)(_sc_body)


def kernel(input_ids, row_pos_from, row_pos_to, col_pos_from, col_pos_to,
           row_table, col_table):
    out = _sc_call(
        input_ids.reshape(_TT * _D),
        row_pos_from.reshape(_TT),
        row_pos_to.reshape(_TT),
        col_pos_from.reshape(_TT),
        col_pos_to.reshape(_TT),
        row_table,
        col_table,
    )
    return out.reshape(_B, _N, _D)


# speedup vs baseline: 2.0559x; 2.0559x over previous
import functools

import jax
import jax.numpy as jnp
from jax import lax
from jax.experimental import pallas as pl
from jax.experimental.pallas import tpu as pltpu
from jax.experimental.pallas import tpu_sc as plsc

_B, _N, _D = 64, 576, 768
_TT = _B * _N
_NC, _NS = 2, 16
_DH = _D // _NC
_TPW = _TT // _NS
_CH = 16
_NCHUNK = _TPW // _CH
_NBUF = 2
_LANES = 16
_VPT = _DH // _LANES
_IDXG = 384
_MAGIC = 8388608.0


def _rne(v):
    return (v + _MAGIC) - _MAGIC


def _sc_body(x_hbm, rpf_hbm, rpt_hbm, cpf_hbm, cpt_hbm, rtab_hbm, ctab_hbm,
             out_hbm,
             rtab_v, ctab_v, pf_v, pt_v, qf_v, qt_v, pidx_v,
             in_v, out_v, in_sem, out_sem):
    c = lax.axis_index("c")
    s = lax.axis_index("s")
    colbase = c * _DH
    tokbase = s * _TPW

    pltpu.sync_copy(rtab_hbm.at[:, pl.ds(colbase, _DH)], rtab_v)
    pltpu.sync_copy(ctab_hbm.at[:, pl.ds(colbase, _DH)], ctab_v)

    def idx_stage(st, carry):
        t0 = tokbase + st * _IDXG
        pltpu.sync_copy(rpf_hbm.at[pl.ds(t0, _IDXG)], pf_v)
        pltpu.sync_copy(rpt_hbm.at[pl.ds(t0, _IDXG)], pt_v)
        pltpu.sync_copy(cpf_hbm.at[pl.ds(t0, _IDXG)], qf_v)
        pltpu.sync_copy(cpt_hbm.at[pl.ds(t0, _IDXG)], qt_v)

        def idx_body(g, carry2):
            sl = pl.ds(g * _LANES, _LANES)
            rf = _rne(pf_v[sl] * 128.0)
            rt = _rne(pt_v[sl] * 128.0)
            cf = _rne(qf_v[sl] * 128.0)
            ct = _rne(qt_v[sl] * 128.0)
            ridx = _rne((rf + rt) * 0.5).astype(jnp.int32)
            cidx = _rne((cf + ct) * 0.5).astype(jnp.int32)
            pidx_v[pl.ds(st * _IDXG + g * _LANES, _LANES)] = (
                (ridx << 8) | cidx)
            return carry2

        return lax.fori_loop(0, _IDXG // _LANES, idx_body, carry, unroll=4)

    lax.fori_loop(0, _TPW // _IDXG, idx_stage, 0)

    def issue_in(b, ch):
        tok0 = tokbase + ch * _CH
        pltpu.async_copy(
            x_hbm.at[pl.ds(tok0, _CH), pl.ds(colbase, _DH)], in_v[b],
            in_sem[b])

    for b in range(_NBUF):
        issue_in(b, b)

    iota16 = lax.iota(jnp.int32, 16)

    def chunk_body(k_i, carry):
        for b in range(_NBUF):
            ch = k_i * _NBUF + b
            tok0 = tokbase + ch * _CH
            pltpu.make_async_copy(
                x_hbm.at[pl.ds(tok0, _CH), pl.ds(colbase, _DH)], in_v[b],
                in_sem[b]).wait()

            @pl.when(ch >= _NBUF)
            def _wait_out():
                pltpu.make_async_copy(
                    out_v[b], out_hbm.at[pl.ds(tok0, _CH),
                                         pl.ds(colbase, _DH)],
                    out_sem[b]).wait()

            def tok_body(t, carry2):
                tvec = jnp.full((_LANES,), ch * _CH + t, jnp.int32)
                packed = plsc.load_gather(pidx_v, [tvec])
                ridx = packed >> 8
                cidx = packed & 255
                for j in range(_VPT):
                    cvec = iota16 + (j * _LANES)
                    sl = pl.ds(j * _LANES, _LANES)
                    r = plsc.load_gather(rtab_v, [ridx, cvec])
                    cc = plsc.load_gather(ctab_v, [cidx, cvec])
                    out_v[b][t, sl] = in_v[b][t, sl] + r + cc
                return carry2

            lax.fori_loop(0, _CH, tok_body, 0)

            pltpu.async_copy(
                out_v[b], out_hbm.at[pl.ds(tok0, _CH), pl.ds(colbase, _DH)],
                out_sem[b])

            @pl.when(ch + _NBUF < _NCHUNK)
            def _prefetch():
                issue_in(b, ch + _NBUF)
        return carry

    lax.fori_loop(0, _NCHUNK // _NBUF, chunk_body, 0)

    for b in range(_NBUF):
        pltpu.make_async_copy(
            out_v[b], out_hbm.at[pl.ds(tokbase, _CH), pl.ds(colbase, _DH)],
            out_sem[b]).wait()


_sc_call = functools.partial(
    pl.kernel,
    out_type=jax.ShapeDtypeStruct((_TT, _D), jnp.float32),
    mesh=plsc.VectorSubcoreMesh(
        core_axis_name="c", subcore_axis_name="s",
        num_cores=_NC, num_subcores=_NS),
    compiler_params=pltpu.CompilerParams(needs_layout_passes=False),
    scratch_types=[
        pltpu.VMEM((128, _DH), jnp.float32),
        pltpu.VMEM((128, _DH), jnp.float32),
        pltpu.VMEM((_IDXG,), jnp.float32),
        pltpu.VMEM((_IDXG,), jnp.float32),
        pltpu.VMEM((_IDXG,), jnp.float32),
        pltpu.VMEM((_IDXG,), jnp.float32),
        pltpu.VMEM((_TPW,), jnp.int32),
        [pltpu.VMEM((_CH, _DH), jnp.float32)] * _NBUF,
        [pltpu.VMEM((_CH, _DH), jnp.float32)] * _NBUF,
        [pltpu.SemaphoreType.DMA] * _NBUF,
        [pltpu.SemaphoreType.DMA] * _NBUF,
    ],
)(_sc_body)


def kernel(input_ids, row_pos_from, row_pos_to, col_pos_from, col_pos_to,
           row_table, col_table):
    out = _sc_call(
        input_ids.reshape(_TT, _D),
        row_pos_from.reshape(_TT),
        row_pos_to.reshape(_TT),
        col_pos_from.reshape(_TT),
        col_pos_to.reshape(_TT),
        row_table,
        col_table,
    )
    return out.reshape(_B, _N, _D)


# speedup vs baseline: 2.3168x; 1.1269x over previous
import functools

import jax
import jax.numpy as jnp
from jax import lax
from jax.experimental import pallas as pl
from jax.experimental.pallas import tpu as pltpu
from jax.experimental.pallas import tpu_sc as plsc

_B, _N, _D = 64, 576, 768
_TT = _B * _N
_NC, _NS = 2, 16
_DH = _D // _NC
_TPW = _TT // _NS
_CH = 16
_NCHUNK = _TPW // _CH
_NBUF = 4
_PDIST = 2
_LANES = 16
_VPT = _DH // _LANES
_IDXG = 384
_MAGIC = 8388608.0


def _rne(v):
    return (v + _MAGIC) - _MAGIC


def _sc_body(x_hbm, rpf_hbm, rpt_hbm, cpf_hbm, cpt_hbm, rtab_hbm, ctab_hbm,
             out_hbm,
             rtab_v, ctab_v, pf_v, pt_v, qf_v, qt_v, pidx_v,
             buf, in_sem, out_sem):
    c = lax.axis_index("c")
    s = lax.axis_index("s")
    colbase = c * _DH
    tokbase = s * _TPW

    pltpu.sync_copy(rtab_hbm.at[:, pl.ds(colbase, _DH)], rtab_v)
    pltpu.sync_copy(ctab_hbm.at[:, pl.ds(colbase, _DH)], ctab_v)

    def idx_stage(st, carry):
        t0 = tokbase + st * _IDXG
        pltpu.sync_copy(rpf_hbm.at[pl.ds(t0, _IDXG)], pf_v)
        pltpu.sync_copy(rpt_hbm.at[pl.ds(t0, _IDXG)], pt_v)
        pltpu.sync_copy(cpf_hbm.at[pl.ds(t0, _IDXG)], qf_v)
        pltpu.sync_copy(cpt_hbm.at[pl.ds(t0, _IDXG)], qt_v)

        def idx_body(g, carry2):
            sl = pl.ds(g * _LANES, _LANES)
            rf = _rne(pf_v[sl] * 128.0)
            rt = _rne(pt_v[sl] * 128.0)
            cf = _rne(qf_v[sl] * 128.0)
            ct = _rne(qt_v[sl] * 128.0)
            ridx = _rne((rf + rt) * 0.5).astype(jnp.int32)
            cidx = _rne((cf + ct) * 0.5).astype(jnp.int32)
            pidx_v[pl.ds(st * _IDXG + g * _LANES, _LANES)] = (
                (ridx << 8) | cidx)
            return carry2

        return lax.fori_loop(0, _IDXG // _LANES, idx_body, carry, unroll=4)

    lax.fori_loop(0, _TPW // _IDXG, idx_stage, 0)

    def in_slice(ch):
        tok0 = tokbase + ch * _CH
        return x_hbm.at[pl.ds(tok0, _CH), pl.ds(colbase, _DH)]

    def out_slice(ch):
        tok0 = tokbase + ch * _CH
        return out_hbm.at[pl.ds(tok0, _CH), pl.ds(colbase, _DH)]

    for p in range(_PDIST):
        pltpu.async_copy(in_slice(p), buf[p], in_sem[p])

    iota16 = lax.iota(jnp.int32, 16)

    def chunk_body(k_i, carry):
        for bb in range(_NBUF):
            ch = k_i * _NBUF + bb
            b = buf[bb]
            pltpu.make_async_copy(in_slice(ch), b, in_sem[bb]).wait()

            bp = (bb + _PDIST) % _NBUF

            @pl.when(ch + _PDIST < _NCHUNK)
            def _prefetch():
                @pl.when(ch >= _NBUF - _PDIST)
                def _wait_out():
                    pltpu.make_async_copy(buf[bp], out_slice(ch), out_sem[bp]
                                          ).wait()

                pltpu.async_copy(in_slice(ch + _PDIST), buf[bp],
                                 in_sem[bp])

            def tok_body(t, carry2):
                tvec = jnp.full((_LANES,), ch * _CH + t, jnp.int32)
                packed = plsc.load_gather(pidx_v, [tvec])
                ridx = packed >> 8
                cidx = packed & 255
                cvec = iota16
                for j in range(_VPT):
                    sl = pl.ds(j * _LANES, _LANES)
                    r = plsc.load_gather(rtab_v, [ridx, cvec])
                    cc = plsc.load_gather(ctab_v, [cidx, cvec])
                    plsc.addupdate(b.at[t, sl], r + cc)
                    if j + 1 < _VPT:
                        cvec = cvec + _LANES
                return carry2

            lax.fori_loop(0, _CH, tok_body, 0, unroll=2)

            pltpu.async_copy(b, out_slice(ch), out_sem[bb])
        return carry

    lax.fori_loop(0, _NCHUNK // _NBUF, chunk_body, 0)

    for bb in range(_NBUF):
        pltpu.make_async_copy(buf[bb], out_slice(bb), out_sem[bb]).wait()


_sc_call = functools.partial(
    pl.kernel,
    out_type=jax.ShapeDtypeStruct((_TT, _D), jnp.float32),
    mesh=plsc.VectorSubcoreMesh(
        core_axis_name="c", subcore_axis_name="s",
        num_cores=_NC, num_subcores=_NS),
    compiler_params=pltpu.CompilerParams(needs_layout_passes=False),
    scratch_types=[
        pltpu.VMEM((128, _DH), jnp.float32),
        pltpu.VMEM((128, _DH), jnp.float32),
        pltpu.VMEM((_IDXG,), jnp.float32),
        pltpu.VMEM((_IDXG,), jnp.float32),
        pltpu.VMEM((_IDXG,), jnp.float32),
        pltpu.VMEM((_IDXG,), jnp.float32),
        pltpu.VMEM((_TPW,), jnp.int32),
        [pltpu.VMEM((_CH, _DH), jnp.float32)] * _NBUF,
        [pltpu.SemaphoreType.DMA] * _NBUF,
        [pltpu.SemaphoreType.DMA] * _NBUF,
    ],
)(_sc_body)


def kernel(input_ids, row_pos_from, row_pos_to, col_pos_from, col_pos_to,
           row_table, col_table):
    out = _sc_call(
        input_ids.reshape(_TT, _D),
        row_pos_from.reshape(_TT),
        row_pos_to.reshape(_TT),
        col_pos_from.reshape(_TT),
        col_pos_to.reshape(_TT),
        row_table,
        col_table,
    )
    return out.reshape(_B, _N, _D)


# speedup vs baseline: 2.4441x; 1.0550x over previous
import functools

import jax
import jax.numpy as jnp
from jax import lax
from jax.experimental import pallas as pl
from jax.experimental.pallas import tpu as pltpu
from jax.experimental.pallas import tpu_sc as plsc

_B, _N, _D = 64, 576, 768
_TT = _B * _N
_NC, _NS = 2, 16
_DH = _D // _NC
_TPW = _TT // _NS
_CH = 16
_NCHUNK = _TPW // _CH
_NBUF = 4
_PDIST = 2
_LANES = 16
_VPT = _DH // _LANES
_IDXG = 384
_MAGIC = 8388608.0


def _rne(v):
    return (v + _MAGIC) - _MAGIC


def _sc_body(x_hbm, rpf_hbm, rpt_hbm, cpf_hbm, cpt_hbm, rtab_hbm, ctab_hbm,
             out_hbm,
             rtab_v, ctab_v, pf_v, pt_v, qf_v, qt_v, pidx_v,
             buf, in_sem, out_sem):
    c = lax.axis_index("c")
    s = lax.axis_index("s")
    colbase = c * _DH
    tokbase = s * _TPW

    pltpu.sync_copy(rtab_hbm.at[:, pl.ds(colbase, _DH)], rtab_v)
    pltpu.sync_copy(ctab_hbm.at[:, pl.ds(colbase, _DH)], ctab_v)

    def idx_stage(st, carry):
        t0 = tokbase + st * _IDXG
        pltpu.sync_copy(rpf_hbm.at[pl.ds(t0, _IDXG)], pf_v)
        pltpu.sync_copy(rpt_hbm.at[pl.ds(t0, _IDXG)], pt_v)
        pltpu.sync_copy(cpf_hbm.at[pl.ds(t0, _IDXG)], qf_v)
        pltpu.sync_copy(cpt_hbm.at[pl.ds(t0, _IDXG)], qt_v)

        def idx_body(g, carry2):
            sl = pl.ds(g * _LANES, _LANES)
            rf = _rne(pf_v[sl] * 128.0)
            rt = _rne(pt_v[sl] * 128.0)
            cf = _rne(qf_v[sl] * 128.0)
            ct = _rne(qt_v[sl] * 128.0)
            ridx = _rne((rf + rt) * 0.5).astype(jnp.int32)
            cidx = _rne((cf + ct) * 0.5).astype(jnp.int32)
            pidx_v[pl.ds(st * _IDXG + g * _LANES, _LANES)] = (
                (ridx << 8) | cidx)
            return carry2

        return lax.fori_loop(0, _IDXG // _LANES, idx_body, carry, unroll=4)

    lax.fori_loop(0, _TPW // _IDXG, idx_stage, 0)

    def in_slice(ch):
        tok0 = tokbase + ch * _CH
        return x_hbm.at[pl.ds(tok0, _CH), pl.ds(colbase, _DH)]

    def out_slice(ch):
        tok0 = tokbase + ch * _CH
        return out_hbm.at[pl.ds(tok0, _CH), pl.ds(colbase, _DH)]

    for p in range(_PDIST):
        pltpu.async_copy(in_slice(p), buf[p], in_sem[p])

    iota16 = lax.iota(jnp.int32, 16)

    def chunk_body(k_i, carry):
        for bb in range(_NBUF):
            ch = k_i * _NBUF + bb
            b = buf[bb]
            pltpu.make_async_copy(in_slice(ch), b, in_sem[bb]).wait()

            bp = (bb + _PDIST) % _NBUF

            @pl.when(ch + _PDIST < _NCHUNK)
            def _prefetch():
                @pl.when(ch >= _NBUF - _PDIST)
                def _wait_out():
                    pltpu.make_async_copy(buf[bp], out_slice(ch), out_sem[bp]
                                          ).wait()

                pltpu.async_copy(in_slice(ch + _PDIST), buf[bp],
                                 in_sem[bp])

            @plsc.parallel_loop(0, _CH, 1, unroll=2)
            def _tok(t):
                tvec = jnp.full((_LANES,), ch * _CH + t, jnp.int32)
                packed = plsc.load_gather(pidx_v, [tvec])
                ridx = packed >> 8
                cidx = packed & 255
                cvec = iota16
                for j in range(_VPT):
                    sl = pl.ds(j * _LANES, _LANES)
                    r = plsc.load_gather(rtab_v, [ridx, cvec])
                    cc = plsc.load_gather(ctab_v, [cidx, cvec])
                    plsc.addupdate(b.at[t, sl], r + cc)
                    if j + 1 < _VPT:
                        cvec = cvec + _LANES

            pltpu.async_copy(b, out_slice(ch), out_sem[bb])
        return carry

    lax.fori_loop(0, _NCHUNK // _NBUF, chunk_body, 0)

    for bb in range(_NBUF):
        pltpu.make_async_copy(buf[bb], out_slice(bb), out_sem[bb]).wait()


_sc_call = functools.partial(
    pl.kernel,
    out_type=jax.ShapeDtypeStruct((_TT, _D), jnp.float32),
    mesh=plsc.VectorSubcoreMesh(
        core_axis_name="c", subcore_axis_name="s",
        num_cores=_NC, num_subcores=_NS),
    compiler_params=pltpu.CompilerParams(needs_layout_passes=False),
    scratch_types=[
        pltpu.VMEM((128, _DH), jnp.float32),
        pltpu.VMEM((128, _DH), jnp.float32),
        pltpu.VMEM((_IDXG,), jnp.float32),
        pltpu.VMEM((_IDXG,), jnp.float32),
        pltpu.VMEM((_IDXG,), jnp.float32),
        pltpu.VMEM((_IDXG,), jnp.float32),
        pltpu.VMEM((_TPW,), jnp.int32),
        [pltpu.VMEM((_CH, _DH), jnp.float32)] * _NBUF,
        [pltpu.SemaphoreType.DMA] * _NBUF,
        [pltpu.SemaphoreType.DMA] * _NBUF,
    ],
)(_sc_body)


def kernel(input_ids, row_pos_from, row_pos_to, col_pos_from, col_pos_to,
           row_table, col_table):
    out = _sc_call(
        input_ids.reshape(_TT, _D),
        row_pos_from.reshape(_TT),
        row_pos_to.reshape(_TT),
        col_pos_from.reshape(_TT),
        col_pos_to.reshape(_TT),
        row_table,
        col_table,
    )
    return out.reshape(_B, _N, _D)


# speedup vs baseline: 4.8774x; 1.9956x over previous
import functools

import jax
import jax.numpy as jnp
from jax import lax
from jax.experimental import pallas as pl
from jax.experimental.pallas import tpu as pltpu
from jax.experimental.pallas import tpu_sc as plsc

_B, _N, _D = 64, 576, 768
_TT = _B * _N
_NC, _NS = 2, 16
_DH = _D // _NC
_TPW = _TT // _NS
_CH = 16
_NCHUNK = _TPW // _CH
_NBUF = 4
_PDIST = 2
_LANES = 16
_VPT = _DH // _LANES
_IDXG = 384
_MAGIC = 8388608.0


def _rne(v):
    return (v + _MAGIC) - _MAGIC


def _sc_body(x_hbm, rpf_hbm, rpt_hbm, cpf_hbm, cpt_hbm, rtab_hbm, ctab_hbm,
             out_hbm,
             rtab_v, ctab_v, pf_v, pt_v, qf_v, qt_v, pidx_v,
             buf, in_sem, out_sem):
    c = lax.axis_index("c")
    s = lax.axis_index("s")
    colbase = c * _DH
    tokbase = s * _TPW

    pltpu.sync_copy(rtab_hbm.at[:, pl.ds(colbase, _DH)], rtab_v)
    pltpu.sync_copy(ctab_hbm.at[:, pl.ds(colbase, _DH)], ctab_v)

    def idx_stage(st, carry):
        t0 = tokbase + st * _IDXG
        pltpu.sync_copy(rpf_hbm.at[pl.ds(t0, _IDXG)], pf_v)
        pltpu.sync_copy(rpt_hbm.at[pl.ds(t0, _IDXG)], pt_v)
        pltpu.sync_copy(cpf_hbm.at[pl.ds(t0, _IDXG)], qf_v)
        pltpu.sync_copy(cpt_hbm.at[pl.ds(t0, _IDXG)], qt_v)

        def idx_body(g, carry2):
            sl = pl.ds(g * _LANES, _LANES)
            rf = _rne(pf_v[sl] * 128.0)
            rt = _rne(pt_v[sl] * 128.0)
            cf = _rne(qf_v[sl] * 128.0)
            ct = _rne(qt_v[sl] * 128.0)
            ridx = _rne((rf + rt) * 0.5).astype(jnp.int32)
            cidx = _rne((cf + ct) * 0.5).astype(jnp.int32)
            pidx_v[pl.ds(st * _IDXG + g * _LANES, _LANES)] = (
                (ridx << 8) | cidx)
            return carry2

        return lax.fori_loop(0, _IDXG // _LANES, idx_body, carry, unroll=4)

    lax.fori_loop(0, _TPW // _IDXG, idx_stage, 0)

    def in_slice(ch):
        tok0 = tokbase + ch * _CH
        return x_hbm.at[pl.ds(tok0, _CH), pl.ds(colbase, _DH)]

    def out_slice(ch):
        tok0 = tokbase + ch * _CH
        return out_hbm.at[pl.ds(tok0, _CH), pl.ds(colbase, _DH)]

    for p in range(_PDIST):
        pltpu.async_copy(in_slice(p), buf[p], in_sem[p])

    iota16 = lax.iota(jnp.int32, 16)

    def chunk_body(k_i, carry):
        for bb in range(_NBUF):
            ch = k_i * _NBUF + bb
            b = buf[bb]
            pltpu.make_async_copy(in_slice(ch), b, in_sem[bb]).wait()

            bp = (bb + _PDIST) % _NBUF

            @pl.when(ch + _PDIST < _NCHUNK)
            def _prefetch():
                @pl.when(ch >= _NBUF - _PDIST)
                def _wait_out():
                    pltpu.make_async_copy(buf[bp], out_slice(ch), out_sem[bp]
                                          ).wait()

                pltpu.async_copy(in_slice(ch + _PDIST), buf[bp],
                                 in_sem[bp])

            @plsc.parallel_loop(0, _CH, 1)
            def _tok(t):
                tvec = jnp.full((_LANES,), ch * _CH + t, jnp.int32)
                packed = plsc.load_gather(pidx_v, [tvec])
                ridx = packed >> 8
                cidx = packed & 255
                cvec = iota16
                sums = []
                for j in range(_VPT):
                    r = plsc.load_gather(rtab_v, [ridx, cvec])
                    cc = plsc.load_gather(ctab_v, [cidx, cvec])
                    sums.append(r + cc)
                    if j + 1 < _VPT:
                        cvec = cvec + _LANES
                for j in range(_VPT):
                    plsc.addupdate(b.at[t, pl.ds(j * _LANES, _LANES)],
                                   sums[j])

            pltpu.async_copy(b, out_slice(ch), out_sem[bb])
        return carry

    lax.fori_loop(0, _NCHUNK // _NBUF, chunk_body, 0)

    for bb in range(_NBUF):
        pltpu.make_async_copy(buf[bb], out_slice(bb), out_sem[bb]).wait()


_sc_call = functools.partial(
    pl.kernel,
    out_type=jax.ShapeDtypeStruct((_TT, _D), jnp.float32),
    mesh=plsc.VectorSubcoreMesh(
        core_axis_name="c", subcore_axis_name="s",
        num_cores=_NC, num_subcores=_NS),
    compiler_params=pltpu.CompilerParams(needs_layout_passes=False),
    scratch_types=[
        pltpu.VMEM((128, _DH), jnp.float32),
        pltpu.VMEM((128, _DH), jnp.float32),
        pltpu.VMEM((_IDXG,), jnp.float32),
        pltpu.VMEM((_IDXG,), jnp.float32),
        pltpu.VMEM((_IDXG,), jnp.float32),
        pltpu.VMEM((_IDXG,), jnp.float32),
        pltpu.VMEM((_TPW,), jnp.int32),
        [pltpu.VMEM((_CH, _DH), jnp.float32)] * _NBUF,
        [pltpu.SemaphoreType.DMA] * _NBUF,
        [pltpu.SemaphoreType.DMA] * _NBUF,
    ],
)(_sc_body)


def kernel(input_ids, row_pos_from, row_pos_to, col_pos_from, col_pos_to,
           row_table, col_table):
    out = _sc_call(
        input_ids.reshape(_TT, _D),
        row_pos_from.reshape(_TT),
        row_pos_to.reshape(_TT),
        col_pos_from.reshape(_TT),
        col_pos_to.reshape(_TT),
        row_table,
        col_table,
    )
    return out.reshape(_B, _N, _D)


# speedup vs baseline: 6.6401x; 1.3614x over previous
import functools

import jax
import jax.numpy as jnp
from jax import lax
from jax.experimental import pallas as pl
from jax.experimental.pallas import tpu as pltpu
from jax.experimental.pallas import tpu_sc as plsc

_B, _N, _D = 64, 576, 768
_TT = _B * _N
_NC, _NS = 2, 16
_DH = _D // _NC
_DHP = _DH // 2
_TPW = _TT // _NS
_CH = 32
_NCHUNK = _TPW // _CH
_NBUF = 4
_PDIST = 2
_LANES = 16
_GPT = _DH // 32
_IDXG = 384
_MAGIC = 8388608.0


def _rne(v):
    return (v + _MAGIC) - _MAGIC


def _pack_table(tab):
    tb = tab.astype(jnp.bfloat16).reshape(128, _D // 32, 2, 16)
    lo = tb[:, :, 0, :]
    hi = tb[:, :, 1, :]
    pairs = jnp.stack([lo, hi], axis=-1)
    packed = jax.lax.bitcast_convert_type(pairs, jnp.int32)
    packed = packed.reshape(128, _D // 2)
    return jnp.stack([packed[:, :_DHP], packed[:, _DHP:]])


def _sc_body(x_hbm, rpf_hbm, rpt_hbm, cpf_hbm, cpt_hbm, rtp_hbm, ctp_hbm,
             out_hbm,
             rtab_v, ctab_v, pf_v, pt_v, qf_v, qt_v, pidx_v,
             buf, in_sem, out_sem):
    c = lax.axis_index("c")
    s = lax.axis_index("s")
    colbase = c * _DH
    tokbase = s * _TPW

    pltpu.sync_copy(rtp_hbm.at[c], rtab_v)
    pltpu.sync_copy(ctp_hbm.at[c], ctab_v)

    def idx_stage(st, carry):
        t0 = tokbase + st * _IDXG
        pltpu.sync_copy(rpf_hbm.at[pl.ds(t0, _IDXG)], pf_v)
        pltpu.sync_copy(rpt_hbm.at[pl.ds(t0, _IDXG)], pt_v)
        pltpu.sync_copy(cpf_hbm.at[pl.ds(t0, _IDXG)], qf_v)
        pltpu.sync_copy(cpt_hbm.at[pl.ds(t0, _IDXG)], qt_v)

        def idx_body(g, carry2):
            sl = pl.ds(g * _LANES, _LANES)
            rf = _rne(pf_v[sl] * 128.0)
            rt = _rne(pt_v[sl] * 128.0)
            cf = _rne(qf_v[sl] * 128.0)
            ct = _rne(qt_v[sl] * 128.0)
            ridx = _rne((rf + rt) * 0.5).astype(jnp.int32)
            cidx = _rne((cf + ct) * 0.5).astype(jnp.int32)
            pidx_v[pl.ds(st * _IDXG + g * _LANES, _LANES)] = (
                (ridx << 8) | cidx)
            return carry2

        return lax.fori_loop(0, _IDXG // _LANES, idx_body, carry, unroll=4)

    lax.fori_loop(0, _TPW // _IDXG, idx_stage, 0)

    def in_slice(ch):
        tok0 = tokbase + ch * _CH
        return x_hbm.at[pl.ds(tok0, _CH), pl.ds(colbase, _DH)]

    def out_slice(ch):
        tok0 = tokbase + ch * _CH
        return out_hbm.at[pl.ds(tok0, _CH), pl.ds(colbase, _DH)]

    for p in range(_PDIST):
        pltpu.async_copy(in_slice(p), buf[p], in_sem[p])

    iota16 = lax.iota(jnp.int32, 16)

    def chunk_body(k_i, carry):
        for bb in range(_NBUF):
            ch = k_i * _NBUF + bb
            b = buf[bb]
            pltpu.make_async_copy(in_slice(ch), b, in_sem[bb]).wait()

            bp = (bb + _PDIST) % _NBUF

            @pl.when(ch + _PDIST < _NCHUNK)
            def _prefetch():
                @pl.when(ch >= _NBUF - _PDIST)
                def _wait_out():
                    pltpu.make_async_copy(buf[bp], out_slice(ch), out_sem[bp]
                                          ).wait()

                pltpu.async_copy(in_slice(ch + _PDIST), buf[bp],
                                 in_sem[bp])

            @plsc.parallel_loop(0, _CH, 1)
            def _tok(t):
                tvec = jnp.full((_LANES,), ch * _CH + t, jnp.int32)
                packed = plsc.load_gather(pidx_v, [tvec])
                ridx = packed >> 8
                cidx = packed & 255
                kvec = iota16
                sums = []
                for j in range(_GPT):
                    gr = plsc.load_gather(rtab_v, [ridx, kvec])
                    gc = plsc.load_gather(ctab_v, [cidx, kvec])
                    rbf = plsc.bitcast(gr, jnp.bfloat16)
                    cbf = plsc.bitcast(gc, jnp.bfloat16)
                    sums.append(rbf + cbf)
                    if j + 1 < _GPT:
                        kvec = kvec + _LANES
                for j in range(_GPT):
                    lo, hi = plsc.unpack(sums[j],
                                         format=plsc.PackFormat.INTERLEAVED)
                    plsc.addupdate(b.at[t, pl.ds(32 * j, _LANES)], lo)
                    plsc.addupdate(b.at[t, pl.ds(32 * j + 16, _LANES)], hi)

            pltpu.async_copy(b, out_slice(ch), out_sem[bb])
        return carry

    lax.fori_loop(0, _NCHUNK // _NBUF, chunk_body, 0)

    for bb in range(_NBUF):
        pltpu.make_async_copy(buf[bb], out_slice(bb), out_sem[bb]).wait()


_sc_call = functools.partial(
    pl.kernel,
    out_type=jax.ShapeDtypeStruct((_TT, _D), jnp.float32),
    mesh=plsc.VectorSubcoreMesh(
        core_axis_name="c", subcore_axis_name="s",
        num_cores=_NC, num_subcores=_NS),
    compiler_params=pltpu.CompilerParams(needs_layout_passes=False),
    scratch_types=[
        pltpu.VMEM((128, _DHP), jnp.int32),
        pltpu.VMEM((128, _DHP), jnp.int32),
        pltpu.VMEM((_IDXG,), jnp.float32),
        pltpu.VMEM((_IDXG,), jnp.float32),
        pltpu.VMEM((_IDXG,), jnp.float32),
        pltpu.VMEM((_IDXG,), jnp.float32),
        pltpu.VMEM((_TPW,), jnp.int32),
        [pltpu.VMEM((_CH, _DH), jnp.float32)] * _NBUF,
        [pltpu.SemaphoreType.DMA] * _NBUF,
        [pltpu.SemaphoreType.DMA] * _NBUF,
    ],
)(_sc_body)


def kernel(input_ids, row_pos_from, row_pos_to, col_pos_from, col_pos_to,
           row_table, col_table):
    out = _sc_call(
        input_ids.reshape(_TT, _D),
        row_pos_from.reshape(_TT),
        row_pos_to.reshape(_TT),
        col_pos_from.reshape(_TT),
        col_pos_to.reshape(_TT),
        _pack_table(row_table),
        _pack_table(col_table),
    )
    return out.reshape(_B, _N, _D)


# speedup vs baseline: 6.6722x; 1.0048x over previous
import functools

import jax
import jax.numpy as jnp
from jax import lax
from jax.experimental import pallas as pl
from jax.experimental.pallas import tpu as pltpu
from jax.experimental.pallas import tpu_sc as plsc

_B, _N, _D = 64, 576, 768
_TT = _B * _N
_NC, _NS = 2, 16
_DH = _D // _NC
_DHP = _DH // 2
_TPW = _TT // _NS
_CH = 24
_NCHUNK = _TPW // _CH
_NBUF = 6
_PDIST = 4
_LANES = 16
_GPT = _DH // 32
_IDXG = 384
_MAGIC = 8388608.0


def _rne(v):
    return (v + _MAGIC) - _MAGIC


def _pack_table(tab):
    tb = tab.astype(jnp.bfloat16).reshape(128, _D // 32, 2, 16)
    lo = tb[:, :, 0, :]
    hi = tb[:, :, 1, :]
    pairs = jnp.stack([lo, hi], axis=-1)
    packed = jax.lax.bitcast_convert_type(pairs, jnp.int32)
    packed = packed.reshape(128, _D // 2)
    return jnp.stack([packed[:, :_DHP], packed[:, _DHP:]])


def _sc_body(x_hbm, rpf_hbm, rpt_hbm, cpf_hbm, cpt_hbm, rtp_hbm, ctp_hbm,
             out_hbm,
             rtab_v, ctab_v, pf_v, pt_v, qf_v, qt_v, pidx_v,
             buf, in_sem, out_sem):
    c = lax.axis_index("c")
    s = lax.axis_index("s")
    colbase = c * _DH
    tokbase = s * _TPW

    pltpu.sync_copy(rtp_hbm.at[c], rtab_v)
    pltpu.sync_copy(ctp_hbm.at[c], ctab_v)

    def idx_stage(st, carry):
        t0 = tokbase + st * _IDXG
        pltpu.sync_copy(rpf_hbm.at[pl.ds(t0, _IDXG)], pf_v)
        pltpu.sync_copy(rpt_hbm.at[pl.ds(t0, _IDXG)], pt_v)
        pltpu.sync_copy(cpf_hbm.at[pl.ds(t0, _IDXG)], qf_v)
        pltpu.sync_copy(cpt_hbm.at[pl.ds(t0, _IDXG)], qt_v)

        @plsc.parallel_loop(0, _IDXG // _LANES, 1)
        def idx_body(g):
            sl = pl.ds(g * _LANES, _LANES)
            rf = _rne(pf_v[sl] * 128.0)
            rt = _rne(pt_v[sl] * 128.0)
            cf = _rne(qf_v[sl] * 128.0)
            ct = _rne(qt_v[sl] * 128.0)
            ridx = _rne((rf + rt) * 0.5).astype(jnp.int32)
            cidx = _rne((cf + ct) * 0.5).astype(jnp.int32)
            pidx_v[pl.ds(st * _IDXG + g * _LANES, _LANES)] = (
                (ridx << 8) | cidx)

        return carry

    lax.fori_loop(0, _TPW // _IDXG, idx_stage, 0)

    def in_slice(ch):
        tok0 = tokbase + ch * _CH
        return x_hbm.at[pl.ds(tok0, _CH), pl.ds(colbase, _DH)]

    def out_slice(ch):
        tok0 = tokbase + ch * _CH
        return out_hbm.at[pl.ds(tok0, _CH), pl.ds(colbase, _DH)]

    for p in range(_PDIST):
        pltpu.async_copy(in_slice(p), buf[p], in_sem[p])

    iota16 = lax.iota(jnp.int32, 16)

    def chunk_body(k_i, carry):
        for bb in range(_NBUF):
            ch = k_i * _NBUF + bb
            b = buf[bb]
            pltpu.make_async_copy(in_slice(ch), b, in_sem[bb]).wait()

            bp = (bb + _PDIST) % _NBUF

            @pl.when(ch + _PDIST < _NCHUNK)
            def _prefetch():
                @pl.when(ch >= _NBUF - _PDIST)
                def _wait_out():
                    pltpu.make_async_copy(buf[bp], out_slice(ch), out_sem[bp]
                                          ).wait()

                pltpu.async_copy(in_slice(ch + _PDIST), buf[bp],
                                 in_sem[bp])

            @plsc.parallel_loop(0, _CH, 1)
            def _tok(t):
                tvec = jnp.full((_LANES,), ch * _CH + t, jnp.int32)
                packed = plsc.load_gather(pidx_v, [tvec])
                ridx = packed >> 8
                cidx = packed & 255
                kvec = iota16
                sums = []
                for j in range(_GPT):
                    gr = plsc.load_gather(rtab_v, [ridx, kvec])
                    gc = plsc.load_gather(ctab_v, [cidx, kvec])
                    rbf = plsc.bitcast(gr, jnp.bfloat16)
                    cbf = plsc.bitcast(gc, jnp.bfloat16)
                    sums.append(rbf + cbf)
                    if j + 1 < _GPT:
                        kvec = kvec + _LANES
                for j in range(_GPT):
                    lo, hi = plsc.unpack(sums[j],
                                         format=plsc.PackFormat.INTERLEAVED)
                    plsc.addupdate(b.at[t, pl.ds(32 * j, _LANES)], lo)
                    plsc.addupdate(b.at[t, pl.ds(32 * j + 16, _LANES)], hi)

            pltpu.async_copy(b, out_slice(ch), out_sem[bb])
        return carry

    lax.fori_loop(0, _NCHUNK // _NBUF, chunk_body, 0)

    for bb in range(_NBUF):
        pltpu.make_async_copy(buf[bb], out_slice(bb), out_sem[bb]).wait()


_sc_call = functools.partial(
    pl.kernel,
    out_type=jax.ShapeDtypeStruct((_TT, _D), jnp.float32),
    mesh=plsc.VectorSubcoreMesh(
        core_axis_name="c", subcore_axis_name="s",
        num_cores=_NC, num_subcores=_NS),
    compiler_params=pltpu.CompilerParams(needs_layout_passes=False),
    scratch_types=[
        pltpu.VMEM((128, _DHP), jnp.int32),
        pltpu.VMEM((128, _DHP), jnp.int32),
        pltpu.VMEM((_IDXG,), jnp.float32),
        pltpu.VMEM((_IDXG,), jnp.float32),
        pltpu.VMEM((_IDXG,), jnp.float32),
        pltpu.VMEM((_IDXG,), jnp.float32),
        pltpu.VMEM((_TPW,), jnp.int32),
        [pltpu.VMEM((_CH, _DH), jnp.float32)] * _NBUF,
        [pltpu.SemaphoreType.DMA] * _NBUF,
        [pltpu.SemaphoreType.DMA] * _NBUF,
    ],
)(_sc_body)


def kernel(input_ids, row_pos_from, row_pos_to, col_pos_from, col_pos_to,
           row_table, col_table):
    out = _sc_call(
        input_ids.reshape(_TT, _D),
        row_pos_from.reshape(_TT),
        row_pos_to.reshape(_TT),
        col_pos_from.reshape(_TT),
        col_pos_to.reshape(_TT),
        _pack_table(row_table),
        _pack_table(col_table),
    )
    return out.reshape(_B, _N, _D)
